# baseline (speedup 1.0000x reference)
import jax
import jax.numpy as jnp
from jax import lax
from jax.experimental import pallas as pl
from jax.experimental.pallas import tpu as pltpu
from jax.experimental.pallas import tpu_sc as plsc

U, P, N, D, C, O = 256, 256, 1024, 128, 256, 128
NC, NS = 2, 16
NW = NC * NS
NSH = 4
NBS = NW // NSH
BSH = U // NBS
SSH = N // NSH
ASH = P // NSH
NEG = -3.0e38


def _shard_pool(coords, c0, npsh, rows, nvec, bnds, acc, cntv):
    (loxA, loyA, lozA, hixA, hiyA, hizA,
     loxB, loyB, lozB, hixB, hiyB, hizB) = bnds
    lanes = lax.iota(jnp.int32, 16)

    def seed_body(s, cnts):
        cntA, cntB = cnts
        xs = jnp.full((16,), coords[0, pl.ds(c0 + s, 16)][0], jnp.float32)
        ys = jnp.full((16,), coords[1, pl.ds(c0 + s, 16)][0], jnp.float32)
        zs = jnp.full((16,), coords[2, pl.ds(c0 + s, 16)][0], jnp.float32)
        mA = ((xs >= loxA) & (hixA >= xs) & (ys >= loyA) & (hiyA >= ys)
              & (zs >= lozA) & (hizA >= zs))
        mB = ((xs >= loxB) & (hixB >= xs) & (ys >= loyB) & (hiyB >= ys)
              & (zs >= lozB) & (hizB >= zs))
        cntA = cntA + mA.astype(jnp.int32)
        cntB = cntB + mB.astype(jnp.int32)
        npA = plsc.all_reduce_population_count(mA)[0]
        npB = plsc.all_reduce_population_count(mB)[0]

        @pl.when(npA + npB > 0)
        def _():
            row = [rows[s, pl.ds(32 * jj, 32)] for jj in range(nvec)]

            def accumulate(npair, mk, grp_off):
                def body(_, mc):
                    lane = plsc.all_reduce_ffs(mc)[0]
                    brow = grp_off + lane
                    for jj in range(nvec):
                        acc[brow, pl.ds(32 * jj, 32)] = jnp.maximum(
                            acc[brow, pl.ds(32 * jj, 32)], row[jj])
                    return mc & (lanes != jnp.full((16,), lane, jnp.int32))

                lax.fori_loop(0, npair, body, mk)

            accumulate(npA, mA, 0)
            accumulate(npB, mB, 16)
        return (cntA, cntB)

    z16 = jnp.zeros((16,), jnp.int32)
    cntA, cntB = lax.fori_loop(0, npsh, seed_body, (z16, z16))
    cntv[pl.ds(0, 16)] = cntA.astype(jnp.float32)
    cntv[pl.ds(16, 16)] = cntB.astype(jnp.float32)


def _sc_pool(ub_hbm, sxyz_hbm, axyz_hbm, sf_hbm, bf_hbm,
             g1p_hbm, g2p_hbm, c1p_hbm, c2p_hbm,
             ubv, sxv, axv, rows1, rows2, acc1, acc2, cntv,
             sem1, sem2):
    wid = lax.axis_index("s") * NC + lax.axis_index("c")
    ish = wid // NBS
    jsh = wid % NBS
    cp1 = pltpu.make_async_copy(sf_hbm.at[pl.ds(ish * SSH, SSH)], rows1, sem1)
    cp1.start()
    cp2 = pltpu.make_async_copy(bf_hbm.at[pl.ds(ish * ASH, ASH)], rows2, sem2)
    cp2.start()
    pltpu.sync_copy(ub_hbm, ubv)
    pltpu.sync_copy(sxyz_hbm, sxv.at[pl.ds(0, 3)])
    pltpu.sync_copy(axyz_hbm, axv.at[pl.ds(0, 3)])
    ball = []
    for g in range(2):
        sl = pl.ds(jsh * BSH + g * 16, 16)
        for d_ in range(3):
            c_ = ubv[d_, sl]
            h_ = ubv[3 + d_, sl] * 0.5
            ball.append(c_ - h_)
            ball.append(c_ + h_)
    (lxA, hxA, lyA, hyA, lzA, hzA, lxB, hxB, lyB, hyB, lzB, hzB) = ball
    ball = (lxA, lyA, lzA, hxA, hyA, hzA, lxB, lyB, lzB, hxB, hyB, hzB)

    def init_body(r, carry):
        for jj in range(C // 32):
            acc1[r, pl.ds(32 * jj, 32)] = jnp.full((32,), 0x03FF, jnp.uint16)
        for jj in range(D // 32):
            acc2[r, pl.ds(32 * jj, 32)] = jnp.full((32,), 0x03FF, jnp.uint16)
        return carry

    lax.fori_loop(0, BSH, init_body, jnp.int32(0))

    cp1.wait()
    _shard_pool(sxv, ish * SSH, SSH, rows1, C // 32, ball, acc1, cntv)
    pltpu.sync_copy(acc1, g1p_hbm.at[pl.ds(wid * BSH, BSH)])
    pltpu.sync_copy(cntv, c1p_hbm.at[pl.ds(wid * BSH, BSH)])

    cp2.wait()
    _shard_pool(axv, ish * ASH, ASH, rows2, D // 32, ball, acc2, cntv)
    pltpu.sync_copy(acc2, g2p_hbm.at[pl.ds(wid * BSH, BSH)])
    pltpu.sync_copy(cntv, c2p_hbm.at[pl.ds(wid * BSH, BSH)])


_sc_pool_call = pl.kernel(
    _sc_pool,
    out_type=[
        jax.ShapeDtypeStruct((NSH * U, C), jnp.uint16),
        jax.ShapeDtypeStruct((NSH * U, D), jnp.uint16),
        jax.ShapeDtypeStruct((NSH * U,), jnp.float32),
        jax.ShapeDtypeStruct((NSH * U,), jnp.float32),
    ],
    mesh=plsc.VectorSubcoreMesh(core_axis_name="c", subcore_axis_name="s",
                                num_cores=NC, num_subcores=NS),
    compiler_params=pltpu.CompilerParams(needs_layout_passes=False,
                                         use_tc_tiling_on_sc=False),
    scratch_types=[
        pltpu.VMEM((6, U), jnp.float32),
        pltpu.VMEM((4, N), jnp.float32),
        pltpu.VMEM((4, P), jnp.float32),
        pltpu.VMEM((SSH, C), jnp.uint16),
        pltpu.VMEM((ASH, D), jnp.uint16),
        pltpu.VMEM((BSH, C), jnp.uint16),
        pltpu.VMEM((BSH, D), jnp.uint16),
        pltpu.VMEM((32,), jnp.float32),
        pltpu.SemaphoreType.DMA,
        pltpu.SemaphoreType.DMA,
    ],
)


def _head_body(g1p_ref, g2p_ref, c1t_ref, c2t_ref, bfu_ref, w_ref, b_ref,
               out_ref):
    def shard_max(ref):
        ki = ref[...].astype(jnp.int32)
        a = jnp.maximum(ki[0:U, :], ki[U:2 * U, :])
        bm = jnp.maximum(ki[2 * U:3 * U, :], ki[3 * U:4 * U, :])
        k = jnp.maximum(a, bm)
        fb = jnp.where(k >= 32768, k - 32768, 65535 - k)
        s = fb >> 15
        e = (fb >> 10) & 31
        m = fb & 1023
        norm = lax.bitcast_convert_type(
            (s << 31) | ((e + 112) << 23) | (m << 13), jnp.float32)
        sub = m.astype(jnp.float32) * jnp.float32(5.9604644775390625e-08)
        sub = jnp.where(s == 1, -sub, sub)
        return jnp.where(e == 0, sub, norm)

    g1 = shard_max(g1p_ref)
    g2 = shard_max(g2p_ref)
    c1 = jnp.sum(c1t_ref[...], axis=1, keepdims=True)
    c2 = jnp.sum(c2t_ref[...], axis=1, keepdims=True)
    g1 = jnp.maximum(g1, jnp.where(c1 < float(N), 0.0, NEG))
    g2 = jnp.maximum(g2, jnp.where(c2 < float(P), 0.0, NEG))
    w = w_ref[...]
    dn = (((1,), (1,)), ((), ()))
    acc = lax.dot_general(g1, w[:, :C], dn, preferred_element_type=jnp.float32)
    acc = acc + lax.dot_general(g2, w[:, C:C + D], dn,
                                preferred_element_type=jnp.float32)
    acc = acc + lax.dot_general(bfu_ref[...], w[:, C + D:], dn,
                                preferred_element_type=jnp.float32)
    a = jnp.abs(acc + b_ref[...] + 1e-6)
    out_ref[...] = a / (1.0 + a)


def kernel(union_box, box_features, agg_xyz, seed_feature, seed_xyz,
           box_feature_union, W, b):
    ub_cols = union_box[0].T
    sxyzT = seed_xyz.T
    axyzT = agg_xyz.T
    def enc(x):
        b = lax.bitcast_convert_type(x.astype(jnp.float16),
                                     jnp.uint16).astype(jnp.int32)
        k = jnp.where(b >= 32768, 65535 - b, b + 32768)
        return k.astype(jnp.uint16)

    g1p, g2p, c1p, c2p = _sc_pool_call(ub_cols, sxyzT, axyzT,
                                       enc(seed_feature).T,
                                       enc(box_features))
    c1t = c1p.reshape(NSH, U).T
    c2t = c2p.reshape(NSH, U).T
    bfu = box_feature_union[:, 0, :]
    out = pl.pallas_call(
        _head_body,
        out_shape=jax.ShapeDtypeStruct((U, O), jnp.float32),
    )(g1p, g2p, c1t, c2t, bfu, W, b.reshape(1, O))
    return out


# speedup vs baseline: 1.0469x; 1.0469x over previous
import jax
import jax.numpy as jnp
from jax import lax
from jax.experimental import pallas as pl
from jax.experimental.pallas import tpu as pltpu
from jax.experimental.pallas import tpu_sc as plsc

U, P, N, D, C, O = 256, 256, 1024, 128, 256, 128
NC, NS = 2, 16
NW = NC * NS
NSH = 4
NBS = NW // NSH
BSH = U // NBS
SSH = N // NSH
ASH = P // NSH
NEG = -3.0e38


def _shard_pool(coords, c0, npsh, rows, nvec, boxcol0, ubv, accref, cntv):
    lanes = lax.iota(jnp.int32, 16)

    def box_body(b, carry):
        col = boxcol0 + b
        sc_ = [ubv[d_, pl.ds(col, 16)][0] for d_ in range(6)]
        lox = jnp.full((16,), sc_[0] - 0.5 * sc_[3], jnp.float32)
        hix = jnp.full((16,), sc_[0] + 0.5 * sc_[3], jnp.float32)
        loy = jnp.full((16,), sc_[1] - 0.5 * sc_[4], jnp.float32)
        hiy = jnp.full((16,), sc_[1] + 0.5 * sc_[4], jnp.float32)
        loz = jnp.full((16,), sc_[2] - 0.5 * sc_[5], jnp.float32)
        hiz = jnp.full((16,), sc_[2] + 0.5 * sc_[5], jnp.float32)
        acc0 = tuple(jnp.full((32,), 0x03FF, jnp.uint16) for _ in range(nvec))

        def sg_body(sg, cnt_acc):
            xv = coords[0, pl.ds(c0 + sg * 16, 16)]
            yv = coords[1, pl.ds(c0 + sg * 16, 16)]
            zv = coords[2, pl.ds(c0 + sg * 16, 16)]
            m = ((xv >= lox) & (hix >= xv) & (yv >= loy) & (hiy >= yv)
                 & (zv >= loz) & (hiz >= zv))
            npair = plsc.all_reduce_population_count(m)[0]

            def walk(_, wc):
                mc = wc[0]
                lane = plsc.all_reduce_ffs(mc)[0]
                srow = sg * 16 + lane
                newav = tuple(
                    jnp.maximum(wc[1 + jj], rows[srow, pl.ds(32 * jj, 32)])
                    for jj in range(nvec))
                return (mc & (lanes != jnp.full((16,), lane, jnp.int32)),
                        *newav)

            wc = lax.fori_loop(0, npair, walk, (m,) + cnt_acc[1:])
            return (cnt_acc[0] + npair, *wc[1:])

        out = lax.fori_loop(0, npsh // 16, sg_body, (jnp.int32(0),) + acc0)
        for jj in range(nvec):
            accref[b, pl.ds(32 * jj, 32)] = out[1 + jj]
        cntv[pl.ds(b, 16)] = jnp.full((16,), out[0].astype(jnp.float32))
        return carry

    lax.fori_loop(0, BSH, box_body, jnp.int32(0))


def _sc_pool(ub_hbm, sxyz_hbm, axyz_hbm, sf_hbm, bf_hbm,
             g1p_hbm, g2p_hbm, c1p_hbm, c2p_hbm,
             ubv, sxv, axv, rows1, rows2, acc1, acc2, cntv,
             sem1, sem2):
    wid = lax.axis_index("s") * NC + lax.axis_index("c")
    ish = wid // NBS
    jsh = wid % NBS
    cp1 = pltpu.make_async_copy(sf_hbm.at[pl.ds(ish * SSH, SSH)], rows1, sem1)
    cp1.start()
    cp2 = pltpu.make_async_copy(bf_hbm.at[pl.ds(ish * ASH, ASH)], rows2, sem2)
    cp2.start()
    pltpu.sync_copy(ub_hbm, ubv)
    pltpu.sync_copy(sxyz_hbm, sxv.at[pl.ds(0, 3)])
    pltpu.sync_copy(axyz_hbm, axv.at[pl.ds(0, 3)])
    cp1.wait()
    _shard_pool(sxv, ish * SSH, SSH, rows1, C // 32, jsh * BSH, ubv,
                acc1, cntv)
    pltpu.sync_copy(acc1, g1p_hbm.at[pl.ds(wid * BSH, BSH)])
    pltpu.sync_copy(cntv.at[pl.ds(0, BSH)],
                    c1p_hbm.at[pl.ds(wid * BSH, BSH)])

    cp2.wait()
    _shard_pool(axv, ish * ASH, ASH, rows2, D // 32, jsh * BSH, ubv,
                acc2, cntv)
    pltpu.sync_copy(acc2, g2p_hbm.at[pl.ds(wid * BSH, BSH)])
    pltpu.sync_copy(cntv.at[pl.ds(0, BSH)],
                    c2p_hbm.at[pl.ds(wid * BSH, BSH)])


_sc_pool_call = pl.kernel(
    _sc_pool,
    out_type=[
        jax.ShapeDtypeStruct((NSH * U, C), jnp.uint16),
        jax.ShapeDtypeStruct((NSH * U, D), jnp.uint16),
        jax.ShapeDtypeStruct((NSH * U,), jnp.float32),
        jax.ShapeDtypeStruct((NSH * U,), jnp.float32),
    ],
    mesh=plsc.VectorSubcoreMesh(core_axis_name="c", subcore_axis_name="s",
                                num_cores=NC, num_subcores=NS),
    compiler_params=pltpu.CompilerParams(needs_layout_passes=False,
                                         use_tc_tiling_on_sc=False),
    scratch_types=[
        pltpu.VMEM((6, U), jnp.float32),
        pltpu.VMEM((4, N), jnp.float32),
        pltpu.VMEM((4, P), jnp.float32),
        pltpu.VMEM((SSH, C), jnp.uint16),
        pltpu.VMEM((ASH, D), jnp.uint16),
        pltpu.VMEM((BSH, C), jnp.uint16),
        pltpu.VMEM((BSH, D), jnp.uint16),
        pltpu.VMEM((BSH + 16,), jnp.float32),
        pltpu.SemaphoreType.DMA,
        pltpu.SemaphoreType.DMA,
    ],
)


def _head_body(g1p_ref, g2p_ref, c1t_ref, c2t_ref, bfu_ref, w_ref, b_ref,
               out_ref):
    def shard_max(ref):
        ki = ref[...].astype(jnp.int32)
        a = jnp.maximum(ki[0:U, :], ki[U:2 * U, :])
        bm = jnp.maximum(ki[2 * U:3 * U, :], ki[3 * U:4 * U, :])
        k = jnp.maximum(a, bm)
        fb = jnp.where(k >= 32768, k - 32768, 65535 - k)
        s = fb >> 15
        e = (fb >> 10) & 31
        m = fb & 1023
        norm = lax.bitcast_convert_type(
            (s << 31) | ((e + 112) << 23) | (m << 13), jnp.float32)
        sub = m.astype(jnp.float32) * jnp.float32(5.9604644775390625e-08)
        sub = jnp.where(s == 1, -sub, sub)
        return jnp.where(e == 0, sub, norm)

    g1 = shard_max(g1p_ref)
    g2 = shard_max(g2p_ref)
    c1 = jnp.sum(c1t_ref[...], axis=1, keepdims=True)
    c2 = jnp.sum(c2t_ref[...], axis=1, keepdims=True)
    g1 = jnp.maximum(g1, jnp.where(c1 < float(N), 0.0, NEG))
    g2 = jnp.maximum(g2, jnp.where(c2 < float(P), 0.0, NEG))
    w = w_ref[...]
    dn = (((1,), (1,)), ((), ()))
    acc = lax.dot_general(g1, w[:, :C], dn, preferred_element_type=jnp.float32)
    acc = acc + lax.dot_general(g2, w[:, C:C + D], dn,
                                preferred_element_type=jnp.float32)
    acc = acc + lax.dot_general(bfu_ref[...], w[:, C + D:], dn,
                                preferred_element_type=jnp.float32)
    a = jnp.abs(acc + b_ref[...] + 1e-6)
    out_ref[...] = a / (1.0 + a)


def kernel(union_box, box_features, agg_xyz, seed_feature, seed_xyz,
           box_feature_union, W, b):
    ub_cols = union_box[0].T
    sxyzT = seed_xyz.T
    axyzT = agg_xyz.T
    def enc(x):
        b = lax.bitcast_convert_type(x.astype(jnp.float16),
                                     jnp.uint16).astype(jnp.int32)
        k = jnp.where(b >= 32768, 65535 - b, b + 32768)
        return k.astype(jnp.uint16)

    g1p, g2p, c1p, c2p = _sc_pool_call(ub_cols, sxyzT, axyzT,
                                       enc(seed_feature).T,
                                       enc(box_features))
    c1t = c1p.reshape(NSH, U).T
    c2t = c2p.reshape(NSH, U).T
    bfu = box_feature_union[:, 0, :]
    out = pl.pallas_call(
        _head_body,
        out_shape=jax.ShapeDtypeStruct((U, O), jnp.float32),
    )(g1p, g2p, c1t, c2t, bfu, W, b.reshape(1, O))
    return out


# speedup vs baseline: 1.1557x; 1.1040x over previous
import jax
import jax.numpy as jnp
from jax import lax
from jax.experimental import pallas as pl
from jax.experimental.pallas import tpu as pltpu
from jax.experimental.pallas import tpu_sc as plsc

U, P, N, D, C, O = 256, 256, 1024, 128, 256, 128
NC, NS = 2, 16
NW = NC * NS
NSH = 4
NBS = NW // NSH
BSH = U // NBS
SSH = N // NSH
ASH = P // NSH
NEG = -3.0e38


def _shard_pool(coords, c0, npsh, rows, nvec, boxcol0, ubv, accref, cntv):
    lanes = lax.iota(jnp.int32, 16)

    def box_body(b, carry):
        col = boxcol0 + b
        sc_ = [ubv[d_, pl.ds(col, 16)][0] for d_ in range(6)]
        lox = jnp.full((16,), sc_[0] - 0.5 * sc_[3], jnp.float32)
        hix = jnp.full((16,), sc_[0] + 0.5 * sc_[3], jnp.float32)
        loy = jnp.full((16,), sc_[1] - 0.5 * sc_[4], jnp.float32)
        hiy = jnp.full((16,), sc_[1] + 0.5 * sc_[4], jnp.float32)
        loz = jnp.full((16,), sc_[2] - 0.5 * sc_[5], jnp.float32)
        hiz = jnp.full((16,), sc_[2] + 0.5 * sc_[5], jnp.float32)
        acc0 = tuple(jnp.full((32,), 0x03FF, jnp.uint16) for _ in range(nvec))

        def sg_body(sg2, cnt_acc):
            ms, nps = [], []
            for h in range(2):
                off = c0 + sg2 * 32 + h * 16
                xv = coords[0, pl.ds(off, 16)]
                yv = coords[1, pl.ds(off, 16)]
                zv = coords[2, pl.ds(off, 16)]
                m = ((xv >= lox) & (hix >= xv) & (yv >= loy) & (hiy >= yv)
                     & (zv >= loz) & (hiz >= zv))
                ms.append(m)
                nps.append(plsc.all_reduce_population_count(m)[0])

            def walk_h(h, carry):
                def walk(_, wc):
                    mc = wc[0]
                    lane = plsc.all_reduce_ffs(mc)[0]
                    srow = sg2 * 32 + h * 16 + lane
                    newav = tuple(
                        jnp.maximum(wc[1 + jj],
                                    rows[srow, pl.ds(32 * jj, 32)])
                        for jj in range(nvec))
                    return (mc & (lanes != jnp.full((16,), lane, jnp.int32)),
                            *newav)

                wc = lax.fori_loop(0, nps[h], walk, (ms[h],) + carry)
                return wc[1:]

            av = cnt_acc[1:]
            av = walk_h(0, av)
            av = walk_h(1, av)
            return (cnt_acc[0] + nps[0] + nps[1], *av)

        out = lax.fori_loop(0, npsh // 32, sg_body, (jnp.int32(0),) + acc0)
        for jj in range(nvec):
            accref[b, pl.ds(32 * jj, 32)] = out[1 + jj]
        cntv[pl.ds(b, 16)] = jnp.full((16,), out[0].astype(jnp.float32))
        return carry

    lax.fori_loop(0, BSH, box_body, jnp.int32(0))


def _sc_pool(ub_hbm, sxyz_hbm, axyz_hbm, sf_hbm, bf_hbm,
             g1p_hbm, g2p_hbm, c1p_hbm, c2p_hbm,
             ubv, sxv, axv, rows1, rows2, acc1, acc2, cntv,
             sem1, sem2):
    wid = lax.axis_index("s") * NC + lax.axis_index("c")
    ish = wid // NBS
    jsh = wid % NBS
    cp1 = pltpu.make_async_copy(sf_hbm.at[pl.ds(ish * SSH, SSH)], rows1, sem1)
    cp1.start()
    cp2 = pltpu.make_async_copy(bf_hbm.at[pl.ds(ish * ASH, ASH)], rows2, sem2)
    cp2.start()
    pltpu.sync_copy(ub_hbm, ubv)
    pltpu.sync_copy(sxyz_hbm, sxv.at[pl.ds(0, 3)])
    pltpu.sync_copy(axyz_hbm, axv.at[pl.ds(0, 3)])
    cp1.wait()
    _shard_pool(sxv, ish * SSH, SSH, rows1, C // 32, jsh * BSH, ubv,
                acc1, cntv)
    pltpu.sync_copy(acc1, g1p_hbm.at[pl.ds(wid * BSH, BSH)])
    pltpu.sync_copy(cntv.at[pl.ds(0, BSH)],
                    c1p_hbm.at[pl.ds(wid * BSH, BSH)])

    cp2.wait()
    _shard_pool(axv, ish * ASH, ASH, rows2, D // 32, jsh * BSH, ubv,
                acc2, cntv)
    pltpu.sync_copy(acc2, g2p_hbm.at[pl.ds(wid * BSH, BSH)])
    pltpu.sync_copy(cntv.at[pl.ds(0, BSH)],
                    c2p_hbm.at[pl.ds(wid * BSH, BSH)])


_sc_pool_call = pl.kernel(
    _sc_pool,
    out_type=[
        jax.ShapeDtypeStruct((NSH * U, C), jnp.uint16),
        jax.ShapeDtypeStruct((NSH * U, D), jnp.uint16),
        jax.ShapeDtypeStruct((NSH * U,), jnp.float32),
        jax.ShapeDtypeStruct((NSH * U,), jnp.float32),
    ],
    mesh=plsc.VectorSubcoreMesh(core_axis_name="c", subcore_axis_name="s",
                                num_cores=NC, num_subcores=NS),
    compiler_params=pltpu.CompilerParams(needs_layout_passes=False,
                                         use_tc_tiling_on_sc=False),
    scratch_types=[
        pltpu.VMEM((6, U), jnp.float32),
        pltpu.VMEM((4, N), jnp.float32),
        pltpu.VMEM((4, P), jnp.float32),
        pltpu.VMEM((SSH, C), jnp.uint16),
        pltpu.VMEM((ASH, D), jnp.uint16),
        pltpu.VMEM((BSH, C), jnp.uint16),
        pltpu.VMEM((BSH, D), jnp.uint16),
        pltpu.VMEM((BSH + 16,), jnp.float32),
        pltpu.SemaphoreType.DMA,
        pltpu.SemaphoreType.DMA,
    ],
)


def _head_body(g1p_ref, g2p_ref, c1t_ref, c2t_ref, bfu_ref, w_ref, b_ref,
               out_ref):
    def shard_max(ref):
        ki = ref[...].astype(jnp.int32)
        a = jnp.maximum(ki[0:U, :], ki[U:2 * U, :])
        bm = jnp.maximum(ki[2 * U:3 * U, :], ki[3 * U:4 * U, :])
        k = jnp.maximum(a, bm)
        fb = jnp.where(k >= 32768, k - 32768, 65535 - k)
        s = fb >> 15
        e = (fb >> 10) & 31
        m = fb & 1023
        norm = lax.bitcast_convert_type(
            (s << 31) | ((e + 112) << 23) | (m << 13), jnp.float32)
        sub = m.astype(jnp.float32) * jnp.float32(5.9604644775390625e-08)
        sub = jnp.where(s == 1, -sub, sub)
        return jnp.where(e == 0, sub, norm)

    g1 = shard_max(g1p_ref)
    g2 = shard_max(g2p_ref)
    c1 = jnp.sum(c1t_ref[...], axis=1, keepdims=True)
    c2 = jnp.sum(c2t_ref[...], axis=1, keepdims=True)
    g1 = jnp.maximum(g1, jnp.where(c1 < float(N), 0.0, NEG))
    g2 = jnp.maximum(g2, jnp.where(c2 < float(P), 0.0, NEG))
    w = w_ref[...]
    dn = (((1,), (1,)), ((), ()))
    acc = lax.dot_general(g1, w[:, :C], dn, preferred_element_type=jnp.float32)
    acc = acc + lax.dot_general(g2, w[:, C:C + D], dn,
                                preferred_element_type=jnp.float32)
    acc = acc + lax.dot_general(bfu_ref[...], w[:, C + D:], dn,
                                preferred_element_type=jnp.float32)
    a = jnp.abs(acc + b_ref[...] + 1e-6)
    out_ref[...] = a / (1.0 + a)


def kernel(union_box, box_features, agg_xyz, seed_feature, seed_xyz,
           box_feature_union, W, b):
    ub_cols = union_box[0].T
    sxyzT = seed_xyz.T
    axyzT = agg_xyz.T
    def enc(x):
        b = lax.bitcast_convert_type(x.astype(jnp.float16),
                                     jnp.uint16).astype(jnp.int32)
        k = jnp.where(b >= 32768, 65535 - b, b + 32768)
        return k.astype(jnp.uint16)

    g1p, g2p, c1p, c2p = _sc_pool_call(ub_cols, sxyzT, axyzT,
                                       enc(seed_feature).T,
                                       enc(box_features))
    c1t = c1p.reshape(NSH, U).T
    c2t = c2p.reshape(NSH, U).T
    bfu = box_feature_union[:, 0, :]
    out = pl.pallas_call(
        _head_body,
        out_shape=jax.ShapeDtypeStruct((U, O), jnp.float32),
    )(g1p, g2p, c1t, c2t, bfu, W, b.reshape(1, O))
    return out


# speedup vs baseline: 1.2176x; 1.0536x over previous
import jax
import jax.numpy as jnp
from jax import lax
from jax.experimental import pallas as pl
from jax.experimental.pallas import tpu as pltpu
from jax.experimental.pallas import tpu_sc as plsc

U, P, N, D, C, O = 256, 256, 1024, 128, 256, 128
NC, NS = 2, 16
NW = NC * NS
NSH = 4
NBS = NW // NSH
BSH = U // NBS
SSH = N // NSH
ASH = P // NSH
NEG = -3.0e38


def _shard_pool(coords, c0, npsh, rows, nvec, boxcol0, ubv, accref, cntv):
    lanes = lax.iota(jnp.int32, 16)

    def box_body(b, carry):
        col = boxcol0 + b
        sc_ = [ubv[d_, pl.ds(col, 16)][0] for d_ in range(6)]
        lox = jnp.full((16,), sc_[0] - 0.5 * sc_[3], jnp.float32)
        hix = jnp.full((16,), sc_[0] + 0.5 * sc_[3], jnp.float32)
        loy = jnp.full((16,), sc_[1] - 0.5 * sc_[4], jnp.float32)
        hiy = jnp.full((16,), sc_[1] + 0.5 * sc_[4], jnp.float32)
        loz = jnp.full((16,), sc_[2] - 0.5 * sc_[5], jnp.float32)
        hiz = jnp.full((16,), sc_[2] + 0.5 * sc_[5], jnp.float32)
        acc0 = tuple(jnp.full((32,), 0x03FF, jnp.uint16) for _ in range(nvec))

        def sg_body(sg2, cnt_acc):
            ms, nps = [], []
            for h in range(4):
                off = c0 + sg2 * 64 + h * 16
                xv = coords[0, pl.ds(off, 16)]
                yv = coords[1, pl.ds(off, 16)]
                zv = coords[2, pl.ds(off, 16)]
                m = ((xv >= lox) & (hix >= xv) & (yv >= loy) & (hiy >= yv)
                     & (zv >= loz) & (hiz >= zv))
                ms.append(m)
                nps.append(plsc.all_reduce_population_count(m)[0])

            def walk_h(h, carry):
                def walk(_, wc):
                    mc = wc[0]
                    lane = plsc.all_reduce_ffs(mc)[0]
                    srow = sg2 * 64 + h * 16 + lane
                    newav = tuple(
                        jnp.maximum(wc[1 + jj],
                                    rows[srow, pl.ds(32 * jj, 32)])
                        for jj in range(nvec))
                    return (mc & (lanes != jnp.full((16,), lane, jnp.int32)),
                            *newav)

                wc = lax.fori_loop(0, nps[h], walk, (ms[h],) + carry)
                return wc[1:]

            av = cnt_acc[1:]
            for h in range(4):
                av = walk_h(h, av)
            return (cnt_acc[0] + nps[0] + nps[1] + nps[2] + nps[3], *av)

        out = lax.fori_loop(0, npsh // 64, sg_body, (jnp.int32(0),) + acc0)
        for jj in range(nvec):
            accref[b, pl.ds(32 * jj, 32)] = out[1 + jj]
        cntv[pl.ds(b, 16)] = jnp.full((16,), out[0].astype(jnp.float32))
        return carry

    lax.fori_loop(0, BSH, box_body, jnp.int32(0))


def _sc_pool(ub_hbm, sxyz_hbm, axyz_hbm, sf_hbm, bf_hbm,
             g1p_hbm, g2p_hbm, c1p_hbm, c2p_hbm,
             ubv, sxv, axv, rows1, rows2, acc1, acc2, cntv,
             sem1, sem2):
    wid = lax.axis_index("s") * NC + lax.axis_index("c")
    ish = wid // NBS
    jsh = wid % NBS
    cp1 = pltpu.make_async_copy(sf_hbm.at[pl.ds(ish * SSH, SSH)], rows1, sem1)
    cp1.start()
    cp2 = pltpu.make_async_copy(bf_hbm.at[pl.ds(ish * ASH, ASH)], rows2, sem2)
    cp2.start()
    pltpu.sync_copy(ub_hbm, ubv)
    pltpu.sync_copy(sxyz_hbm, sxv.at[pl.ds(0, 3)])
    pltpu.sync_copy(axyz_hbm, axv.at[pl.ds(0, 3)])
    cp1.wait()
    _shard_pool(sxv, ish * SSH, SSH, rows1, C // 32, jsh * BSH, ubv,
                acc1, cntv)
    pltpu.sync_copy(acc1, g1p_hbm.at[pl.ds(wid * BSH, BSH)])
    pltpu.sync_copy(cntv.at[pl.ds(0, BSH)],
                    c1p_hbm.at[pl.ds(wid * BSH, BSH)])

    cp2.wait()
    _shard_pool(axv, ish * ASH, ASH, rows2, D // 32, jsh * BSH, ubv,
                acc2, cntv)
    pltpu.sync_copy(acc2, g2p_hbm.at[pl.ds(wid * BSH, BSH)])
    pltpu.sync_copy(cntv.at[pl.ds(0, BSH)],
                    c2p_hbm.at[pl.ds(wid * BSH, BSH)])


_sc_pool_call = pl.kernel(
    _sc_pool,
    out_type=[
        jax.ShapeDtypeStruct((NSH * U, C), jnp.uint16),
        jax.ShapeDtypeStruct((NSH * U, D), jnp.uint16),
        jax.ShapeDtypeStruct((NSH * U,), jnp.float32),
        jax.ShapeDtypeStruct((NSH * U,), jnp.float32),
    ],
    mesh=plsc.VectorSubcoreMesh(core_axis_name="c", subcore_axis_name="s",
                                num_cores=NC, num_subcores=NS),
    compiler_params=pltpu.CompilerParams(needs_layout_passes=False,
                                         use_tc_tiling_on_sc=False),
    scratch_types=[
        pltpu.VMEM((6, U), jnp.float32),
        pltpu.VMEM((4, N), jnp.float32),
        pltpu.VMEM((4, P), jnp.float32),
        pltpu.VMEM((SSH, C), jnp.uint16),
        pltpu.VMEM((ASH, D), jnp.uint16),
        pltpu.VMEM((BSH, C), jnp.uint16),
        pltpu.VMEM((BSH, D), jnp.uint16),
        pltpu.VMEM((BSH + 16,), jnp.float32),
        pltpu.SemaphoreType.DMA,
        pltpu.SemaphoreType.DMA,
    ],
)


def _head_body(g1p_ref, g2p_ref, c1t_ref, c2t_ref, bfu_ref, w_ref, b_ref,
               out_ref):
    def shard_max(ref):
        ki = ref[...].astype(jnp.int32)
        a = jnp.maximum(ki[0:U, :], ki[U:2 * U, :])
        bm = jnp.maximum(ki[2 * U:3 * U, :], ki[3 * U:4 * U, :])
        k = jnp.maximum(a, bm)
        fb = jnp.where(k >= 32768, k - 32768, 65535 - k)
        s = fb >> 15
        e = (fb >> 10) & 31
        m = fb & 1023
        norm = lax.bitcast_convert_type(
            (s << 31) | ((e + 112) << 23) | (m << 13), jnp.float32)
        sub = m.astype(jnp.float32) * jnp.float32(5.9604644775390625e-08)
        sub = jnp.where(s == 1, -sub, sub)
        return jnp.where(e == 0, sub, norm)

    g1 = shard_max(g1p_ref)
    g2 = shard_max(g2p_ref)
    c1 = jnp.sum(c1t_ref[...], axis=1, keepdims=True)
    c2 = jnp.sum(c2t_ref[...], axis=1, keepdims=True)
    g1 = jnp.maximum(g1, jnp.where(c1 < float(N), 0.0, NEG))
    g2 = jnp.maximum(g2, jnp.where(c2 < float(P), 0.0, NEG))
    w = w_ref[...]
    dn = (((1,), (1,)), ((), ()))
    acc = lax.dot_general(g1, w[:, :C], dn, preferred_element_type=jnp.float32)
    acc = acc + lax.dot_general(g2, w[:, C:C + D], dn,
                                preferred_element_type=jnp.float32)
    acc = acc + lax.dot_general(bfu_ref[...], w[:, C + D:], dn,
                                preferred_element_type=jnp.float32)
    a = jnp.abs(acc + b_ref[...] + 1e-6)
    out_ref[...] = a / (1.0 + a)


def kernel(union_box, box_features, agg_xyz, seed_feature, seed_xyz,
           box_feature_union, W, b):
    ub_cols = union_box[0].T
    sxyzT = seed_xyz.T
    axyzT = agg_xyz.T
    def enc(x):
        b = lax.bitcast_convert_type(x.astype(jnp.float16),
                                     jnp.uint16).astype(jnp.int32)
        k = jnp.where(b >= 32768, 65535 - b, b + 32768)
        return k.astype(jnp.uint16)

    g1p, g2p, c1p, c2p = _sc_pool_call(ub_cols, sxyzT, axyzT,
                                       enc(seed_feature).T,
                                       enc(box_features))
    c1t = c1p.reshape(NSH, U).T
    c2t = c2p.reshape(NSH, U).T
    bfu = box_feature_union[:, 0, :]
    out = pl.pallas_call(
        _head_body,
        out_shape=jax.ShapeDtypeStruct((U, O), jnp.float32),
    )(g1p, g2p, c1t, c2t, bfu, W, b.reshape(1, O))
    return out


# speedup vs baseline: 1.4164x; 1.1632x over previous
import jax
import jax.numpy as jnp
from jax import lax
from jax.experimental import pallas as pl
from jax.experimental.pallas import tpu as pltpu
from jax.experimental.pallas import tpu_sc as plsc

U, P, N, D, C, O = 256, 256, 1024, 128, 256, 128
NC, NS = 2, 16
NW = NC * NS
NSH = 4
NBS = NW // NSH
BSH = U // NBS
SSH = N // NSH
ASH = P // NSH
NEG = -3.0e38


def _shard_pool(coords, c0, npsh, rows, nvec, boxcol0, ubv, accref, cntv):
    lanes = lax.iota(jnp.int32, 16)

    def box_body(b, carry):
        col = boxcol0 + b
        sc_ = [ubv[d_, pl.ds(col, 16)][0] for d_ in range(6)]
        lox = jnp.full((16,), sc_[0] - 0.5 * sc_[3], jnp.float32)
        hix = jnp.full((16,), sc_[0] + 0.5 * sc_[3], jnp.float32)
        loy = jnp.full((16,), sc_[1] - 0.5 * sc_[4], jnp.float32)
        hiy = jnp.full((16,), sc_[1] + 0.5 * sc_[4], jnp.float32)
        loz = jnp.full((16,), sc_[2] - 0.5 * sc_[5], jnp.float32)
        hiz = jnp.full((16,), sc_[2] + 0.5 * sc_[5], jnp.float32)
        acc0 = tuple(jnp.full((32,), 0x03FF, jnp.uint16) for _ in range(nvec))

        def sg_body(sg2, cnt_acc):
            ms, nps = [], []
            for h in range(4):
                off = c0 + sg2 * 64 + h * 16
                xv = coords[0, pl.ds(off, 16)]
                yv = coords[1, pl.ds(off, 16)]
                zv = coords[2, pl.ds(off, 16)]
                m = ((xv >= lox) & (hix >= xv) & (yv >= loy) & (hiy >= yv)
                     & (zv >= loz) & (hiz >= zv))
                ms.append(m)
                nps.append(plsc.all_reduce_population_count(m)[0])

            def walk_h(h, carry):
                def walk(_, wc):
                    mc = wc[0]
                    lane = plsc.all_reduce_ffs(mc)[0]
                    srow = sg2 * 64 + h * 16 + lane
                    newav = tuple(
                        jnp.maximum(wc[1 + jj],
                                    rows[srow, pl.ds(32 * jj, 32)])
                        for jj in range(nvec))
                    return (mc & (lanes != jnp.full((16,), lane, jnp.int32)),
                            *newav)

                wc = lax.fori_loop(0, nps[h], walk, (ms[h],) + carry)
                return wc[1:]

            av = cnt_acc[1:]
            for h in range(4):
                av = walk_h(h, av)
            return (cnt_acc[0] + nps[0] + nps[1] + nps[2] + nps[3], *av)

        out = lax.fori_loop(0, npsh // 64, sg_body, (jnp.int32(0),) + acc0)
        for jj in range(nvec):
            accref[b, pl.ds(32 * jj, 32)] = out[1 + jj]
        cntv[pl.ds(b, 16)] = jnp.full((16,), out[0].astype(jnp.float32))
        return carry

    lax.fori_loop(0, BSH, box_body, jnp.int32(0))


def _sc_pool(ub_hbm, sxyz_hbm, sf_hbm, g1p_hbm, c1p_hbm,
             ubv, sxv, rows1, acc1, cntv, sem1):
    wid = lax.axis_index("s") * NC + lax.axis_index("c")
    ish = wid // NBS
    jsh = wid % NBS
    cp1 = pltpu.make_async_copy(sf_hbm.at[pl.ds(ish * SSH, SSH)], rows1, sem1)
    cp1.start()
    pltpu.sync_copy(ub_hbm, ubv)
    pltpu.sync_copy(sxyz_hbm, sxv.at[pl.ds(0, 3)])
    cp1.wait()
    _shard_pool(sxv, ish * SSH, SSH, rows1, C // 32, jsh * BSH, ubv,
                acc1, cntv)
    pltpu.sync_copy(acc1, g1p_hbm.at[pl.ds(wid * BSH, BSH)])
    pltpu.sync_copy(cntv.at[pl.ds(0, BSH)],
                    c1p_hbm.at[pl.ds(wid * BSH, BSH)])


_sc_pool_call = pl.kernel(
    _sc_pool,
    out_type=[
        jax.ShapeDtypeStruct((NSH * U, C), jnp.uint16),
        jax.ShapeDtypeStruct((NSH * U,), jnp.float32),
    ],
    mesh=plsc.VectorSubcoreMesh(core_axis_name="c", subcore_axis_name="s",
                                num_cores=NC, num_subcores=NS),
    compiler_params=pltpu.CompilerParams(needs_layout_passes=False,
                                         use_tc_tiling_on_sc=False),
    scratch_types=[
        pltpu.VMEM((6, U), jnp.float32),
        pltpu.VMEM((4, N), jnp.float32),
        pltpu.VMEM((SSH, C), jnp.uint16),
        pltpu.VMEM((BSH, C), jnp.uint16),
        pltpu.VMEM((BSH + 16,), jnp.float32),
        pltpu.SemaphoreType.DMA,
    ],
)


AGG_UB = 8


def _agg_pool_body(ub_ref, axyz_ref, bf_ref, g2_ref):
    ubb = ub_ref[...]
    cmin = ubb[:, 0:3] - 0.5 * ubb[:, 3:6]
    cmax = ubb[:, 0:3] + 0.5 * ubb[:, 3:6]
    ax = axyz_ref[0:1, :]
    ay = axyz_ref[1:2, :]
    az = axyz_ref[2:3, :]
    bf = bf_ref[...]
    ma = ((ax >= cmin[:, 0:1]) & (cmax[:, 0:1] >= ax)
          & (ay >= cmin[:, 1:2]) & (cmax[:, 1:2] >= ay)
          & (az >= cmin[:, 2:3]) & (cmax[:, 2:3] >= az))
    for b in range(AGG_UB):
        t2 = jnp.where(ma[b:b + 1, :], bf, 0.0)
        g2_ref[b, :, :] = jnp.max(t2, axis=1, keepdims=True).reshape(1, D)


def _head_body(g1p_ref, g2_ref, c1t_ref, bfu_ref, w_ref, b_ref,
               out_ref):
    def shard_max(ref):
        ki = ref[...].astype(jnp.int32)
        a = jnp.maximum(ki[0:U, :], ki[U:2 * U, :])
        bm = jnp.maximum(ki[2 * U:3 * U, :], ki[3 * U:4 * U, :])
        k = jnp.maximum(a, bm)
        fb = jnp.where(k >= 32768, k - 32768, 65535 - k)
        s = fb >> 15
        e = (fb >> 10) & 31
        m = fb & 1023
        norm = lax.bitcast_convert_type(
            (s << 31) | ((e + 112) << 23) | (m << 13), jnp.float32)
        sub = m.astype(jnp.float32) * jnp.float32(5.9604644775390625e-08)
        sub = jnp.where(s == 1, -sub, sub)
        return jnp.where(e == 0, sub, norm)

    g1 = shard_max(g1p_ref)
    g2 = g2_ref[...]
    c1 = jnp.sum(c1t_ref[...], axis=1, keepdims=True)
    g1 = jnp.maximum(g1, jnp.where(c1 < float(N), 0.0, NEG))
    w = w_ref[...]
    dn = (((1,), (1,)), ((), ()))
    acc = lax.dot_general(g1, w[:, :C], dn, preferred_element_type=jnp.float32)
    acc = acc + lax.dot_general(g2, w[:, C:C + D], dn,
                                preferred_element_type=jnp.float32)
    acc = acc + lax.dot_general(bfu_ref[...], w[:, C + D:], dn,
                                preferred_element_type=jnp.float32)
    a = jnp.abs(acc + b_ref[...] + 1e-6)
    out_ref[...] = a / (1.0 + a)


def kernel(union_box, box_features, agg_xyz, seed_feature, seed_xyz,
           box_feature_union, W, b):
    ub_cols = union_box[0].T
    sxyzT = seed_xyz.T
    axyzT = agg_xyz.T
    def enc(x):
        b = lax.bitcast_convert_type(x.astype(jnp.float16),
                                     jnp.uint16).astype(jnp.int32)
        k = jnp.where(b >= 32768, 65535 - b, b + 32768)
        return k.astype(jnp.uint16)

    g1p, c1p = _sc_pool_call(ub_cols, sxyzT, enc(seed_feature).T)
    ubpad = jnp.pad(union_box[0], ((0, 0), (0, 2)))
    bfT = box_features.astype(jnp.float16).astype(jnp.float32).T
    g2 = pl.pallas_call(
        _agg_pool_body,
        grid=(U // AGG_UB,),
        in_specs=[
            pl.BlockSpec((AGG_UB, 8), lambda u: (u, 0)),
            pl.BlockSpec((8, P), lambda u: (0, 0)),
            pl.BlockSpec((D, P), lambda u: (0, 0)),
        ],
        out_specs=pl.BlockSpec((AGG_UB, 1, D), lambda u: (u, 0, 0)),
        out_shape=jax.ShapeDtypeStruct((U, 1, D), jnp.float32),
    )(ubpad, jnp.pad(axyzT, ((0, 5), (0, 0))), bfT).reshape(U, D)
    c1t = c1p.reshape(NSH, U).T
    bfu = box_feature_union[:, 0, :]
    out = pl.pallas_call(
        _head_body,
        out_shape=jax.ShapeDtypeStruct((U, O), jnp.float32),
    )(g1p, g2, c1t, bfu, W, b.reshape(1, O))
    return out
